# baseline (speedup 1.0000x reference)
import functools

import jax
import jax.numpy as jnp
from jax import lax
from jax.experimental import pallas as pl
from jax.experimental.pallas import tpu as pltpu
from jax.experimental.pallas import tpu_sc as plsc

_TOTAL = 32768
_B = 16
_D = 128
_NC = 2
_NS = 16
_LANES = 16
_ROWS_PER_W = _TOTAL // (_NC * _NS)
_CHUNK = 128
_NCHUNK = _ROWS_PER_W // _CHUNK


def _sc_body(x_hbm, seg_hbm, sums_out, cnts_out,
             xbuf, idxbuf, onesbuf, zbuf, acc_sh, cnt_sh):
    c = lax.axis_index("c")
    s = lax.axis_index("s")
    base = (c * _NS + s) * _ROWS_PER_W

    ones_v = jnp.ones((_LANES,), jnp.float32)
    zero_v = jnp.zeros((_LANES,), jnp.float32)
    for i in range(_CHUNK):
        for j in range(_D // _LANES):
            onesbuf[i, pl.ds(j * _LANES, _LANES)] = ones_v

    @pl.when(s == 0)
    def _zero_shared():
        for i in range(_B):
            for j in range(_D // _LANES):
                zbuf[i, pl.ds(j * _LANES, _LANES)] = zero_v
        pltpu.sync_copy(zbuf, acc_sh)
        pltpu.sync_copy(zbuf, cnt_sh)

    plsc.subcore_barrier()

    for g in range(_NCHUNK):
        off = base + g * _CHUNK
        pltpu.sync_copy(x_hbm.at[pl.ds(off, _CHUNK)], xbuf)
        pltpu.sync_copy(seg_hbm.at[pl.ds(off, _CHUNK)], idxbuf)
        pltpu.sync_copy(xbuf, acc_sh.at[idxbuf], add=True)
        pltpu.sync_copy(onesbuf, cnt_sh.at[idxbuf], add=True)

    plsc.subcore_barrier()

    @pl.when(s == 0)
    def _flush():
        pltpu.sync_copy(acc_sh, sums_out.at[c])
        pltpu.sync_copy(cnt_sh, cnts_out.at[c])


_sc_segment_sums = functools.partial(
    pl.kernel,
    out_type=[jax.ShapeDtypeStruct((_NC, _B, _D), jnp.float32),
              jax.ShapeDtypeStruct((_NC, _B, _D), jnp.float32)],
    mesh=plsc.VectorSubcoreMesh(core_axis_name="c", subcore_axis_name="s",
                                num_cores=_NC, num_subcores=_NS),
    scratch_types=[
        pltpu.VMEM((_CHUNK, _D), jnp.float32),
        pltpu.VMEM((_CHUNK,), jnp.int32),
        pltpu.VMEM((_CHUNK, _D), jnp.float32),
        pltpu.VMEM((_B, _D), jnp.float32),
        pltpu.VMEM_SHARED((_B, _D), jnp.float32),
        pltpu.VMEM_SHARED((_B, _D), jnp.float32),
    ],
)(_sc_body)


def _tc_finish(sums_ref, cnts_ref, wenc_ref, benc_ref, wq0_ref, wq1_ref,
               keys_ref, q0_ref, q1_ref):
    s = sums_ref[0] + sums_ref[1]
    cnt = cnts_ref[0] + cnts_ref[1]
    denom = jnp.maximum(cnt, 1.0)
    keys = (jnp.dot(s, wenc_ref[...], preferred_element_type=jnp.float32)
            + cnt * benc_ref[...]) / denom
    keys_ref[...] = keys
    q0_ref[...] = jnp.dot(keys, wq0_ref[...], preferred_element_type=jnp.float32)
    q1_ref[...] = jnp.dot(keys, wq1_ref[...], preferred_element_type=jnp.float32)


def kernel(x, segment_ids, W_enc, b_enc, W_q0, W_q1):
    sums, cnts = _sc_segment_sums(x, segment_ids)
    keys, q0, q1 = pl.pallas_call(
        _tc_finish,
        out_shape=[jax.ShapeDtypeStruct((_B, _D), jnp.float32)] * 3,
    )(sums, cnts, W_enc, b_enc.reshape(1, _D), W_q0, W_q1)
    return (keys, q0, q1)


# speedup vs baseline: 1.3351x; 1.3351x over previous
import functools

import jax
import jax.numpy as jnp
from jax import lax
from jax.experimental import pallas as pl
from jax.experimental.pallas import tpu as pltpu
from jax.experimental.pallas import tpu_sc as plsc

_TOTAL = 32768
_B = 16
_D = 128
_NC = 2
_NS = 16
_LANES = 16
_ROWS_PER_W = _TOTAL // (_NC * _NS)
_CHUNK = 128
_NCHUNK = _ROWS_PER_W // _CHUNK


def _sc_body(x_hbm, seg_hbm, sums_out, cnts_out,
             xbuf, idxbuf, onesbuf, zbuf, acc_sh, cnt_sh, lsem, ssem, osem):
    c = lax.axis_index("c")
    s = lax.axis_index("s")
    w = c * _NS + s
    base = w * _ROWS_PER_W

    ones_v = jnp.ones((_LANES,), jnp.float32)
    zero_v = jnp.zeros((_LANES,), jnp.float32)

    def _fill_ones(i, carry):
        for j in range(_D // _LANES):
            onesbuf[i, pl.ds(j * _LANES, _LANES)] = ones_v
        return carry

    lax.fori_loop(0, _CHUNK, _fill_ones, 0)

    pltpu.sync_copy(seg_hbm.at[pl.ds(w * _NCHUNK, _NCHUNK)], idxbuf)

    @pl.when(s == 0)
    def _zero_shared():
        for i in range(_B):
            for j in range(_D // _LANES):
                zbuf[i, pl.ds(j * _LANES, _LANES)] = zero_v
        pltpu.sync_copy(zbuf, acc_sh)
        pltpu.sync_copy(zbuf, cnt_sh)

    plsc.subcore_barrier()

    ld = [None] * _NCHUNK
    sc = [None] * _NCHUNK
    on = [None] * _NCHUNK
    ld[0] = pltpu.async_copy(x_hbm.at[pl.ds(base, _CHUNK)], xbuf.at[0],
                             lsem.at[0])
    for g in range(_NCHUNK):
        b = g % 2
        ld[g].wait()
        sc[g] = pltpu.async_copy(xbuf.at[b], acc_sh.at[idxbuf.at[g]],
                                 ssem.at[b], add=True)
        on[g] = pltpu.async_copy(onesbuf, cnt_sh.at[idxbuf.at[g]],
                                 osem, add=True)
        if g + 1 < _NCHUNK:
            if g >= 1:
                sc[g - 1].wait()
            ld[g + 1] = pltpu.async_copy(
                x_hbm.at[pl.ds(base + (g + 1) * _CHUNK, _CHUNK)],
                xbuf.at[1 - b], lsem.at[1 - b])
    sc[_NCHUNK - 1].wait()
    for g in range(_NCHUNK):
        on[g].wait()

    plsc.subcore_barrier()

    @pl.when(s == 0)
    def _flush():
        pltpu.sync_copy(acc_sh, sums_out.at[c])
        pltpu.sync_copy(cnt_sh, cnts_out.at[c])


_sc_segment_sums = functools.partial(
    pl.kernel,
    out_type=[jax.ShapeDtypeStruct((_NC, _B, _D), jnp.float32),
              jax.ShapeDtypeStruct((_NC, _B, _D), jnp.float32)],
    mesh=plsc.VectorSubcoreMesh(core_axis_name="c", subcore_axis_name="s",
                                num_cores=_NC, num_subcores=_NS),
    scratch_types=[
        pltpu.VMEM((2, _CHUNK, _D), jnp.float32),
        pltpu.VMEM((_NCHUNK, _CHUNK), jnp.int32),
        pltpu.VMEM((_CHUNK, _D), jnp.float32),
        pltpu.VMEM((_B, _D), jnp.float32),
        pltpu.VMEM_SHARED((_B, _D), jnp.float32),
        pltpu.VMEM_SHARED((_B, _D), jnp.float32),
        pltpu.SemaphoreType.DMA((2,)),
        pltpu.SemaphoreType.DMA((2,)),
        pltpu.SemaphoreType.DMA,
    ],
)(_sc_body)


def _tc_finish(sums_ref, cnts_ref, wenc_ref, benc_ref, wq0_ref, wq1_ref,
               keys_ref, q0_ref, q1_ref):
    s = sums_ref[0] + sums_ref[1]
    cnt = cnts_ref[0] + cnts_ref[1]
    denom = jnp.maximum(cnt, 1.0)
    keys = (jnp.dot(s, wenc_ref[...], preferred_element_type=jnp.float32)
            + cnt * benc_ref[...]) / denom
    keys_ref[...] = keys
    q0_ref[...] = jnp.dot(keys, wq0_ref[...], preferred_element_type=jnp.float32)
    q1_ref[...] = jnp.dot(keys, wq1_ref[...], preferred_element_type=jnp.float32)


def kernel(x, segment_ids, W_enc, b_enc, W_q0, W_q1):
    seg2 = segment_ids.reshape(_TOTAL // _CHUNK, _CHUNK)
    sums, cnts = _sc_segment_sums(x, seg2)
    keys, q0, q1 = pl.pallas_call(
        _tc_finish,
        out_shape=[jax.ShapeDtypeStruct((_B, _D), jnp.float32)] * 3,
    )(sums, cnts, W_enc, b_enc.reshape(1, _D), W_q0, W_q1)
    return (keys, q0, q1)


# speedup vs baseline: 1.4651x; 1.0974x over previous
import functools

import jax
import jax.numpy as jnp
from jax import lax
from jax.experimental import pallas as pl
from jax.experimental.pallas import tpu as pltpu
from jax.experimental.pallas import tpu_sc as plsc

_TOTAL = 32768
_B = 16
_D = 128
_NC = 2
_NS = 16
_LANES = 16
_ROWS_PER_W = _TOTAL // (_NC * _NS)
_CHUNK = 128
_NCHUNK = _ROWS_PER_W // _CHUNK
_CBLK = 4096
_NCBLK = _TOTAL // _CBLK


def _sc_body(x_hbm, seg_hbm, sums_out, xbuf, idxbuf, zbuf, acc_sh,
             lsem, ssem):
    c = lax.axis_index("c")
    s = lax.axis_index("s")
    w = c * _NS + s
    base = w * _ROWS_PER_W

    zero_v = jnp.zeros((_LANES,), jnp.float32)

    pltpu.sync_copy(seg_hbm.at[pl.ds(w * _NCHUNK, _NCHUNK)], idxbuf)

    @pl.when(s == 0)
    def _zero_shared():
        for i in range(_B):
            for j in range(_D // _LANES):
                zbuf[i, pl.ds(j * _LANES, _LANES)] = zero_v
        pltpu.sync_copy(zbuf, acc_sh)

    plsc.subcore_barrier()

    ld = [None] * _NCHUNK
    sc = [None] * _NCHUNK
    ld[0] = pltpu.async_copy(x_hbm.at[pl.ds(base, _CHUNK)], xbuf.at[0],
                             lsem.at[0])
    for g in range(_NCHUNK):
        b = g % 2
        ld[g].wait()
        sc[g] = pltpu.async_copy(xbuf.at[b], acc_sh.at[idxbuf.at[g]],
                                 ssem.at[b], add=True)
        if g + 1 < _NCHUNK:
            if g >= 1:
                sc[g - 1].wait()
            ld[g + 1] = pltpu.async_copy(
                x_hbm.at[pl.ds(base + (g + 1) * _CHUNK, _CHUNK)],
                xbuf.at[1 - b], lsem.at[1 - b])
    sc[_NCHUNK - 2].wait()
    sc[_NCHUNK - 1].wait()

    plsc.subcore_barrier()

    @pl.when(s == 0)
    def _flush():
        pltpu.sync_copy(acc_sh, sums_out.at[c])


_sc_segment_sums = functools.partial(
    pl.kernel,
    out_type=jax.ShapeDtypeStruct((_NC, _B, _D), jnp.float32),
    mesh=plsc.VectorSubcoreMesh(core_axis_name="c", subcore_axis_name="s",
                                num_cores=_NC, num_subcores=_NS),
    scratch_types=[
        pltpu.VMEM((2, _CHUNK, _D), jnp.float32),
        pltpu.VMEM((_NCHUNK, _CHUNK), jnp.int32),
        pltpu.VMEM((_B, _D), jnp.float32),
        pltpu.VMEM_SHARED((_B, _D), jnp.float32),
        pltpu.SemaphoreType.DMA((2,)),
        pltpu.SemaphoreType.DMA((2,)),
    ],
)(_sc_body)


def _tc_counts(seg_ref, cnt_ref, acc_ref):
    i = pl.program_id(0)

    @pl.when(i == 0)
    def _init():
        acc_ref[...] = jnp.zeros((_B, _D), jnp.float32)

    seg = seg_ref[0]
    onehot = (seg[:, :, None] == lax.broadcasted_iota(jnp.int32, (1, _CBLK, _B), 2)
              ).astype(jnp.float32)[0]
    acc_ref[...] += lax.dot_general(onehot, jnp.ones((_CBLK, _D), jnp.float32),
                                    (((0,), (0,)), ((), ())),
                                    preferred_element_type=jnp.float32)

    @pl.when(i == _NCBLK - 1)
    def _out():
        cnt_ref[...] = acc_ref[...]


def _tc_finish(sums_ref, cnts_ref, wenc_ref, benc_ref, wq0_ref, wq1_ref,
               keys_ref, q0_ref, q1_ref):
    s = sums_ref[0] + sums_ref[1]
    cnt = cnts_ref[...]
    denom = jnp.maximum(cnt, 1.0)
    keys = (jnp.dot(s, wenc_ref[...], preferred_element_type=jnp.float32)
            + cnt * benc_ref[...]) / denom
    keys_ref[...] = keys
    q0_ref[...] = jnp.dot(keys, wq0_ref[...], preferred_element_type=jnp.float32)
    q1_ref[...] = jnp.dot(keys, wq1_ref[...], preferred_element_type=jnp.float32)


def kernel(x, segment_ids, W_enc, b_enc, W_q0, W_q1):
    seg2 = segment_ids.reshape(_TOTAL // _CHUNK, _CHUNK)
    sums = _sc_segment_sums(x, seg2)
    seg3 = segment_ids.reshape(_NCBLK, 1, _CBLK)
    cnts = pl.pallas_call(
        _tc_counts,
        grid=(_NCBLK,),
        in_specs=[pl.BlockSpec((1, 1, _CBLK), lambda i: (i, 0, 0))],
        out_specs=pl.BlockSpec((_B, _D), lambda i: (0, 0)),
        out_shape=jax.ShapeDtypeStruct((_B, _D), jnp.float32),
        scratch_shapes=[pltpu.VMEM((_B, _D), jnp.float32)],
        compiler_params=pltpu.CompilerParams(
            dimension_semantics=("arbitrary",)),
    )(seg3)
    keys, q0, q1 = pl.pallas_call(
        _tc_finish,
        out_shape=[jax.ShapeDtypeStruct((_B, _D), jnp.float32)] * 3,
    )(sums, cnts, W_enc, b_enc.reshape(1, _D), W_q0, W_q1)
    return (keys, q0, q1)


# speedup vs baseline: 1.5631x; 1.0669x over previous
import functools

import jax
import jax.numpy as jnp
from jax import lax
from jax.experimental import pallas as pl
from jax.experimental.pallas import tpu as pltpu
from jax.experimental.pallas import tpu_sc as plsc

_TOTAL = 32768
_B = 16
_D = 128
_NC = 2
_NS = 16
_LANES = 16
_ROWS_PER_W = _TOTAL // (_NC * _NS)
_CHUNK = 128
_NCHUNK = _ROWS_PER_W // _CHUNK
_NBUF = 4
_CBLK = 4096
_NCBLK = _TOTAL // _CBLK


def _sc_body(x_hbm, seg_hbm, sums_out, xbuf, idxbuf, zbuf, acc_sh,
             lsem, ssem):
    c = lax.axis_index("c")
    s = lax.axis_index("s")
    w = c * _NS + s
    base = w * _ROWS_PER_W

    zero_v = jnp.zeros((_LANES,), jnp.float32)

    pltpu.sync_copy(seg_hbm.at[pl.ds(w * _NCHUNK, _NCHUNK)], idxbuf)

    @pl.when(s == 0)
    def _zero_shared():
        for i in range(_B):
            for j in range(_D // _LANES):
                zbuf[i, pl.ds(j * _LANES, _LANES)] = zero_v
        pltpu.sync_copy(zbuf, acc_sh)

    plsc.subcore_barrier()

    ld = [None] * _NCHUNK
    sc = [None] * _NCHUNK
    for g in range(_NBUF - 1):
        ld[g] = pltpu.async_copy(x_hbm.at[pl.ds(base + g * _CHUNK, _CHUNK)],
                                 xbuf.at[g % _NBUF], lsem.at[g % _NBUF])
    for g in range(_NCHUNK):
        b = g % _NBUF
        ld[g].wait()
        sc[g] = pltpu.async_copy(xbuf.at[b], acc_sh.at[idxbuf.at[g]],
                                 ssem.at[b], add=True)
        nxt = g + _NBUF - 1
        if nxt < _NCHUNK:
            if g >= 1:
                sc[g - 1].wait()
            ld[nxt] = pltpu.async_copy(
                x_hbm.at[pl.ds(base + nxt * _CHUNK, _CHUNK)],
                xbuf.at[nxt % _NBUF], lsem.at[nxt % _NBUF])
    for g in range(max(0, _NCHUNK - _NBUF), _NCHUNK):
        sc[g].wait()

    plsc.subcore_barrier()

    @pl.when(s == 0)
    def _flush():
        pltpu.sync_copy(acc_sh, sums_out.at[c])


_sc_segment_sums = functools.partial(
    pl.kernel,
    out_type=jax.ShapeDtypeStruct((_NC, _B, _D), jnp.float32),
    mesh=plsc.VectorSubcoreMesh(core_axis_name="c", subcore_axis_name="s",
                                num_cores=_NC, num_subcores=_NS),
    scratch_types=[
        pltpu.VMEM((_NBUF, _CHUNK, _D), jnp.float32),
        pltpu.VMEM((_NCHUNK, _CHUNK), jnp.int32),
        pltpu.VMEM((_B, _D), jnp.float32),
        pltpu.VMEM_SHARED((_B, _D), jnp.float32),
        pltpu.SemaphoreType.DMA((_NBUF,)),
        pltpu.SemaphoreType.DMA((_NBUF,)),
    ],
)(_sc_body)


def _tc_counts(seg_ref, cnt_ref, acc_ref):
    i = pl.program_id(0)

    @pl.when(i == 0)
    def _init():
        acc_ref[...] = jnp.zeros((_B, _D), jnp.float32)

    seg = seg_ref[0]
    onehot = (seg[:, :, None] == lax.broadcasted_iota(jnp.int32, (1, _CBLK, _B), 2)
              ).astype(jnp.float32)[0]
    acc_ref[...] += lax.dot_general(onehot, jnp.ones((_CBLK, _D), jnp.float32),
                                    (((0,), (0,)), ((), ())),
                                    preferred_element_type=jnp.float32)

    @pl.when(i == _NCBLK - 1)
    def _out():
        cnt_ref[...] = acc_ref[...]


def _tc_finish(sums_ref, cnts_ref, wenc_ref, benc_ref, wq0_ref, wq1_ref,
               keys_ref, q0_ref, q1_ref):
    s = sums_ref[0] + sums_ref[1]
    cnt = cnts_ref[...]
    denom = jnp.maximum(cnt, 1.0)
    keys = (jnp.dot(s, wenc_ref[...], preferred_element_type=jnp.float32)
            + cnt * benc_ref[...]) / denom
    keys_ref[...] = keys
    q0_ref[...] = jnp.dot(keys, wq0_ref[...], preferred_element_type=jnp.float32)
    q1_ref[...] = jnp.dot(keys, wq1_ref[...], preferred_element_type=jnp.float32)


def kernel(x, segment_ids, W_enc, b_enc, W_q0, W_q1):
    seg2 = segment_ids.reshape(_TOTAL // _CHUNK, _CHUNK)
    seg3 = segment_ids.reshape(_NCBLK, 1, _CBLK)
    cnts = pl.pallas_call(
        _tc_counts,
        grid=(_NCBLK,),
        in_specs=[pl.BlockSpec((1, 1, _CBLK), lambda i: (i, 0, 0))],
        out_specs=pl.BlockSpec((_B, _D), lambda i: (0, 0)),
        out_shape=jax.ShapeDtypeStruct((_B, _D), jnp.float32),
        scratch_shapes=[pltpu.VMEM((_B, _D), jnp.float32)],
        compiler_params=pltpu.CompilerParams(
            dimension_semantics=("arbitrary",)),
    )(seg3)
    sums = _sc_segment_sums(x, seg2)
    keys, q0, q1 = pl.pallas_call(
        _tc_finish,
        out_shape=[jax.ShapeDtypeStruct((_B, _D), jnp.float32)] * 3,
    )(sums, cnts, W_enc, b_enc.reshape(1, _D), W_q0, W_q1)
    return (keys, q0, q1)
